# baseline (speedup 1.0000x reference)
import functools

import jax
import jax.numpy as jnp
from jax import lax
from jax.experimental import pallas as pl
from jax.experimental.pallas import tpu as pltpu
from jax.experimental.pallas import tpu_sc as plsc

TOKEN_TILE = 128
_INTERPRET = False


def _gate_kernel(x_ref, gw_ref, gb_ref,
                 e1_ref, e2_ref, w1_ref, w2_ref, r1_ref, r2_ref, cnt_ref):
    i = pl.program_id(0)

    @pl.when(i == 0)
    def _():
        cnt_ref[...] = jnp.zeros_like(cnt_ref)

    xt = x_ref[...]
    logits = jnp.dot(xt, gw_ref[...], preferred_element_type=jnp.float32)
    logits = logits + gb_ref[...]
    m = jnp.max(logits, axis=1, keepdims=True)
    p = jnp.exp(logits - m)
    probs = p / jnp.sum(p, axis=1, keepdims=True)

    t, e = probs.shape
    eio = lax.broadcasted_iota(jnp.int32, (t, e), 1)
    g1 = jnp.max(probs, axis=1, keepdims=True)
    e1 = jnp.min(jnp.where(probs == g1, eio, e), axis=1, keepdims=True)
    oh1 = eio == e1
    probs2 = jnp.where(oh1, -1.0, probs)
    g2 = jnp.max(probs2, axis=1, keepdims=True)
    e2 = jnp.min(jnp.where(probs2 == g2, eio, e), axis=1, keepdims=True)
    oh2 = eio == e2

    denom = g1 + g2
    oh1f = oh1.astype(jnp.float32)
    oh2f = oh2.astype(jnp.float32)

    row = lax.broadcasted_iota(jnp.int32, (t, t), 0)
    col = lax.broadcasted_iota(jnp.int32, (t, t), 1)
    tril = (row > col).astype(jnp.float32)
    cnt1 = jnp.dot(tril, oh1f, preferred_element_type=jnp.float32)
    cnt2 = jnp.dot(tril, oh2f, preferred_element_type=jnp.float32)

    base = cnt_ref[...]
    r1 = jnp.sum((base + cnt1) * oh1f, axis=1, keepdims=True)
    tc1 = jnp.sum(oh1f, axis=0, keepdims=True)
    r2 = jnp.sum((base + tc1 + cnt2) * oh2f, axis=1, keepdims=True)
    cnt_ref[...] = base + tc1 + jnp.sum(oh2f, axis=0, keepdims=True)

    e1_ref[...] = e1.astype(jnp.int32)[None]
    e2_ref[...] = e2.astype(jnp.int32)[None]
    w1_ref[...] = (g1 / denom)[None]
    w2_ref[...] = (g2 / denom)[None]
    r1_ref[...] = r1.astype(jnp.int32)[None]
    r2_ref[...] = r2.astype(jnp.int32)[None]


def _gating(x2, gate_W, gate_b):
    s, h = x2.shape
    e = gate_W.shape[1]
    nt = s // TOKEN_TILE
    out_shapes = (
        jax.ShapeDtypeStruct((nt, TOKEN_TILE, 1), jnp.int32),
        jax.ShapeDtypeStruct((nt, TOKEN_TILE, 1), jnp.int32),
        jax.ShapeDtypeStruct((nt, TOKEN_TILE, 1), jnp.float32),
        jax.ShapeDtypeStruct((nt, TOKEN_TILE, 1), jnp.float32),
        jax.ShapeDtypeStruct((nt, TOKEN_TILE, 1), jnp.int32),
        jax.ShapeDtypeStruct((nt, TOKEN_TILE, 1), jnp.int32),
        jax.ShapeDtypeStruct((1, e), jnp.float32),
    )
    tile3 = pl.BlockSpec((1, TOKEN_TILE, 1), lambda i: (i, 0, 0))
    outs = pl.pallas_call(
        _gate_kernel,
        grid=(nt,),
        in_specs=[
            pl.BlockSpec((TOKEN_TILE, h), lambda i: (i, 0)),
            pl.BlockSpec((h, e), lambda i: (0, 0)),
            pl.BlockSpec((1, e), lambda i: (0, 0)),
        ],
        out_specs=(tile3, tile3, tile3, tile3, tile3, tile3,
                   pl.BlockSpec((1, e), lambda i: (0, 0))),
        out_shape=out_shapes,
        interpret=_INTERPRET,
    )(x2, gate_W, gate_b.reshape(1, e))
    e1, e2, w1, w2, r1, r2, counts = outs
    flat = lambda a: a.reshape(s)
    return (flat(e1), flat(e2), flat(w1), flat(w2), flat(r1), flat(r2),
            counts.reshape(e).astype(jnp.int32))


_GATHER_CHUNK = 256


def _expert_kernel(te_ref, ntl_ref, x_ref, rt_ref, w1a_ref, w1b_ref, b1_ref,
                   w2a_ref, w2b_ref, b2_ref, g_ref, ys_ref):
    t = pl.program_id(0)

    @pl.when(t < ntl_ref[0])
    def _():
        s, hh = x_ref.shape
        f2 = w1a_ref.shape[2]
        rid = rt_ref[0]
        xg = jnp.zeros((TOKEN_TILE, hh), jnp.float32)
        for kc in range(0, s, _GATHER_CHUNK):
            ik = kc + lax.broadcasted_iota(
                jnp.int32, (TOKEN_TILE, _GATHER_CHUNK), 1)
            p = (rid == ik).astype(jnp.float32)
            xg += jnp.dot(p, x_ref[kc:kc + _GATHER_CHUNK, :],
                          preferred_element_type=jnp.float32)
        h1 = jnp.dot(xg, w1a_ref[0], preferred_element_type=jnp.float32)
        h1 = jnp.maximum(h1 + b1_ref[0, :, :f2], 0.0)
        h2 = jnp.dot(xg, w1b_ref[0], preferred_element_type=jnp.float32)
        h2 = jnp.maximum(h2 + b1_ref[0, :, f2:], 0.0)
        y = jnp.dot(h1, w2a_ref[0], preferred_element_type=jnp.float32)
        y = y + jnp.dot(h2, w2b_ref[0], preferred_element_type=jnp.float32)
        ys_ref[...] = (y + b2_ref[0]) * g_ref[...]


def _grouped_experts(x2, row_token3, W1, b1, W2, b2, gates, tile_expert,
                     ntiles, max_tiles):
    s, h = x2.shape
    e, _, f = W1.shape
    f2 = f // 2
    p = max_tiles * TOKEN_TILE
    grid_spec = pltpu.PrefetchScalarGridSpec(
        num_scalar_prefetch=2,
        grid=(max_tiles,),
        in_specs=[
            pl.BlockSpec((s, h), lambda t, te, ntl: (0, 0)),
            pl.BlockSpec((1, TOKEN_TILE, 1),
                         lambda t, te, ntl: (jnp.minimum(t, ntl[0] - 1), 0, 0)),
            pl.BlockSpec((1, h, f2), lambda t, te, ntl: (te[t], 0, 0)),
            pl.BlockSpec((1, h, f2), lambda t, te, ntl: (te[t], 0, 1)),
            pl.BlockSpec((1, 1, f), lambda t, te, ntl: (te[t], 0, 0)),
            pl.BlockSpec((1, f2, h), lambda t, te, ntl: (te[t], 0, 0)),
            pl.BlockSpec((1, f2, h), lambda t, te, ntl: (te[t], 1, 0)),
            pl.BlockSpec((1, 1, h), lambda t, te, ntl: (te[t], 0, 0)),
            pl.BlockSpec((TOKEN_TILE, 1),
                         lambda t, te, ntl: (jnp.minimum(t, ntl[0] - 1), 0)),
        ],
        out_specs=pl.BlockSpec(
            (TOKEN_TILE, h),
            lambda t, te, ntl: (jnp.minimum(t, ntl[0] - 1), 0)),
    )
    return pl.pallas_call(
        _expert_kernel,
        grid_spec=grid_spec,
        out_shape=jax.ShapeDtypeStruct((p, h), jnp.float32),
        interpret=_INTERPRET,
    )(tile_expert, ntiles, x2, row_token3, W1, W1, b1.reshape(e, 1, f), W2,
      W2, b2.reshape(e, 1, h), gates.reshape(p, 1))


def _combine_sc(ys, pos1, pos2, s):
    _, h = ys.shape
    info = plsc.get_sparse_core_info()
    nw = info.num_cores * info.num_subcores
    nl = info.num_lanes
    chunk = s // nw
    mesh = plsc.VectorSubcoreMesh(core_axis_name="c", subcore_axis_name="s")

    @functools.partial(
        pl.kernel, mesh=mesh,
        out_type=jax.ShapeDtypeStruct((s, h), jnp.float32),
        scratch_types=[
            pltpu.VMEM((chunk, h), jnp.float32),
            pltpu.VMEM((chunk, h), jnp.float32),
            pltpu.VMEM((chunk,), jnp.int32),
            pltpu.VMEM((chunk,), jnp.int32),
            pltpu.SemaphoreType.DMA,
            pltpu.SemaphoreType.DMA,
        ],
    )
    def k(ys_hbm, p1_hbm, p2_hbm, out_hbm,
          r1_v, r2_v, i1_v, i2_v, sem1, sem2):
        wid = lax.axis_index("s") * info.num_cores + lax.axis_index("c")
        base = wid * chunk
        pltpu.sync_copy(p1_hbm.at[pl.ds(base, chunk)], i1_v)
        pltpu.sync_copy(p2_hbm.at[pl.ds(base, chunk)], i2_v)
        c1 = pltpu.async_copy(ys_hbm.at[i1_v], r1_v, sem1)
        c2 = pltpu.async_copy(ys_hbm.at[i2_v], r2_v, sem2)
        c1.wait()
        c2.wait()

        def token_body(i, _):
            for j in range(h // nl):
                sl = pl.ds(j * nl, nl)
                r1_v[i, sl] = r1_v[i, sl] + r2_v[i, sl]
            return 0

        lax.fori_loop(0, chunk, token_body, 0)
        pltpu.sync_copy(r1_v, out_hbm.at[pl.ds(base, chunk)])

    return k(ys, pos1, pos2)


def kernel(x, gate_W, gate_b, W1, b1, W2, b2):
    b, s, h = x.shape
    e = W1.shape[0]
    x2 = x.reshape(b * s, h)
    st = b * s

    e1, e2, w1, w2, r1, r2, counts = _gating(x2, gate_W, gate_b)

    nt_e = (counts + TOKEN_TILE - 1) // TOKEN_TILE
    cum = jnp.cumsum(nt_e)
    ntiles = cum[-1]
    tile_base = (cum - nt_e) * TOKEN_TILE
    max_tiles = (2 * st) // TOKEN_TILE + e - 1
    tq = jnp.minimum(jnp.arange(max_tiles, dtype=jnp.int32), ntiles - 1)
    tile_expert = jnp.searchsorted(cum, tq, side="right").astype(jnp.int32)
    pos1 = tile_base[e1] + r1
    pos2 = tile_base[e2] + r2
    num_rows = max_tiles * TOKEN_TILE
    ar = jnp.arange(st, dtype=jnp.int32)
    row_token = (jnp.zeros((num_rows,), jnp.int32)
                 .at[pos1].set(ar).at[pos2].set(ar))
    gates = (jnp.zeros((num_rows,), jnp.float32)
             .at[pos1].set(w1).at[pos2].set(w2))

    ys = _grouped_experts(x2, row_token.reshape(max_tiles, TOKEN_TILE, 1),
                          W1, b1, W2, b2, gates, tile_expert,
                          ntiles.reshape(1).astype(jnp.int32), max_tiles)
    out = _combine_sc(ys, pos1, pos2, st)
    return out.reshape(b, s, h)


# speedup vs baseline: 1.0699x; 1.0699x over previous
import functools

import jax
import jax.numpy as jnp
from jax import lax
from jax.experimental import pallas as pl
from jax.experimental.pallas import tpu as pltpu
from jax.experimental.pallas import tpu_sc as plsc

TOKEN_TILE = 128
_INTERPRET = False


def _gate_kernel(x_ref, gw_ref, gb_ref,
                 e1_ref, e2_ref, w1_ref, w2_ref, r1_ref, r2_ref, cnt_ref):
    i = pl.program_id(0)

    @pl.when(i == 0)
    def _():
        cnt_ref[...] = jnp.zeros_like(cnt_ref)

    xt = x_ref[...]
    logits = jnp.dot(xt, gw_ref[...], preferred_element_type=jnp.float32)
    logits = logits + gb_ref[...]
    m = jnp.max(logits, axis=1, keepdims=True)
    p = jnp.exp(logits - m)
    probs = p / jnp.sum(p, axis=1, keepdims=True)

    t, e = probs.shape
    eio = lax.broadcasted_iota(jnp.int32, (t, e), 1)
    g1 = jnp.max(probs, axis=1, keepdims=True)
    e1 = jnp.min(jnp.where(probs == g1, eio, e), axis=1, keepdims=True)
    oh1 = eio == e1
    probs2 = jnp.where(oh1, -1.0, probs)
    g2 = jnp.max(probs2, axis=1, keepdims=True)
    e2 = jnp.min(jnp.where(probs2 == g2, eio, e), axis=1, keepdims=True)
    oh2 = eio == e2

    denom = g1 + g2
    oh1f = oh1.astype(jnp.float32)
    oh2f = oh2.astype(jnp.float32)

    row = lax.broadcasted_iota(jnp.int32, (t, t), 0)
    col = lax.broadcasted_iota(jnp.int32, (t, t), 1)
    tril = (row > col).astype(jnp.float32)
    cnt1 = jnp.dot(tril, oh1f, preferred_element_type=jnp.float32)
    cnt2 = jnp.dot(tril, oh2f, preferred_element_type=jnp.float32)

    base = cnt_ref[...]
    r1 = jnp.sum((base + cnt1) * oh1f, axis=1, keepdims=True)
    tc1 = jnp.sum(oh1f, axis=0, keepdims=True)
    r2 = jnp.sum((base + tc1 + cnt2) * oh2f, axis=1, keepdims=True)
    cnt_ref[...] = base + tc1 + jnp.sum(oh2f, axis=0, keepdims=True)

    e1_ref[...] = e1.astype(jnp.int32)[None]
    e2_ref[...] = e2.astype(jnp.int32)[None]
    w1_ref[...] = (g1 / denom)[None]
    w2_ref[...] = (g2 / denom)[None]
    r1_ref[...] = r1.astype(jnp.int32)[None]
    r2_ref[...] = r2.astype(jnp.int32)[None]


def _gating(x2, gate_W, gate_b):
    s, h = x2.shape
    e = gate_W.shape[1]
    nt = s // TOKEN_TILE
    out_shapes = (
        jax.ShapeDtypeStruct((nt, TOKEN_TILE, 1), jnp.int32),
        jax.ShapeDtypeStruct((nt, TOKEN_TILE, 1), jnp.int32),
        jax.ShapeDtypeStruct((nt, TOKEN_TILE, 1), jnp.float32),
        jax.ShapeDtypeStruct((nt, TOKEN_TILE, 1), jnp.float32),
        jax.ShapeDtypeStruct((nt, TOKEN_TILE, 1), jnp.int32),
        jax.ShapeDtypeStruct((nt, TOKEN_TILE, 1), jnp.int32),
        jax.ShapeDtypeStruct((1, e), jnp.float32),
    )
    tile3 = pl.BlockSpec((1, TOKEN_TILE, 1), lambda i: (i, 0, 0))
    outs = pl.pallas_call(
        _gate_kernel,
        grid=(nt,),
        in_specs=[
            pl.BlockSpec((TOKEN_TILE, h), lambda i: (i, 0)),
            pl.BlockSpec((h, e), lambda i: (0, 0)),
            pl.BlockSpec((1, e), lambda i: (0, 0)),
        ],
        out_specs=(tile3, tile3, tile3, tile3, tile3, tile3,
                   pl.BlockSpec((1, e), lambda i: (0, 0))),
        out_shape=out_shapes,
        interpret=_INTERPRET,
    )(x2, gate_W, gate_b.reshape(1, e))
    e1, e2, w1, w2, r1, r2, counts = outs
    flat = lambda a: a.reshape(s)
    return (flat(e1), flat(e2), flat(w1), flat(w2), flat(r1), flat(r2),
            counts.reshape(e).astype(jnp.int32))


def _expert_kernel(te_ref, ntl_ref, xs_ref, w1_ref, b1_ref, w2_ref, b2_ref,
                   g_ref, ys_ref):
    t = pl.program_id(0)

    @pl.when(t < ntl_ref[0])
    def _():
        h = jnp.dot(xs_ref[...], w1_ref[0], preferred_element_type=jnp.float32)
        h = jnp.maximum(h + b1_ref[0], 0.0)
        y = jnp.dot(h, w2_ref[0], preferred_element_type=jnp.float32)
        ys_ref[...] = (y + b2_ref[0]) * g_ref[...]


def _grouped_experts(xs, W1, b1, W2, b2, gates, tile_expert, ntiles):
    p, h = xs.shape
    e, _, f = W1.shape
    max_tiles = p // TOKEN_TILE
    grid_spec = pltpu.PrefetchScalarGridSpec(
        num_scalar_prefetch=2,
        grid=(max_tiles,),
        in_specs=[
            pl.BlockSpec((TOKEN_TILE, h),
                         lambda t, te, ntl: (jnp.minimum(t, ntl[0] - 1), 0)),
            pl.BlockSpec((1, h, f), lambda t, te, ntl: (te[t], 0, 0)),
            pl.BlockSpec((1, 1, f), lambda t, te, ntl: (te[t], 0, 0)),
            pl.BlockSpec((1, f, h), lambda t, te, ntl: (te[t], 0, 0)),
            pl.BlockSpec((1, 1, h), lambda t, te, ntl: (te[t], 0, 0)),
            pl.BlockSpec((TOKEN_TILE, 1),
                         lambda t, te, ntl: (jnp.minimum(t, ntl[0] - 1), 0)),
        ],
        out_specs=pl.BlockSpec(
            (TOKEN_TILE, h),
            lambda t, te, ntl: (jnp.minimum(t, ntl[0] - 1), 0)),
    )
    return pl.pallas_call(
        _expert_kernel,
        grid_spec=grid_spec,
        out_shape=jax.ShapeDtypeStruct((p, h), jnp.float32),
        interpret=_INTERPRET,
    )(tile_expert, ntiles, xs, W1, b1.reshape(e, 1, f), W2,
      b2.reshape(e, 1, h), gates.reshape(p, 1))


def _dispatch_sc(x2, pos1, pos2, num_rows):
    s, h = x2.shape
    info = plsc.get_sparse_core_info()
    nw = info.num_cores * info.num_subcores
    chunk = s // nw
    mesh = plsc.VectorSubcoreMesh(core_axis_name="c", subcore_axis_name="s")

    @functools.partial(
        pl.kernel, mesh=mesh,
        out_type=jax.ShapeDtypeStruct((num_rows, h), jnp.float32),
        scratch_types=[
            pltpu.VMEM((chunk, h), jnp.float32),
            pltpu.VMEM((chunk,), jnp.int32),
            pltpu.VMEM((chunk,), jnp.int32),
            pltpu.SemaphoreType.DMA,
            pltpu.SemaphoreType.DMA,
        ],
    )
    def k(x_hbm, p1_hbm, p2_hbm, xs_hbm, rows_v, i1_v, i2_v, sem1, sem2):
        wid = lax.axis_index("s") * info.num_cores + lax.axis_index("c")
        base = wid * chunk
        pltpu.sync_copy(x_hbm.at[pl.ds(base, chunk)], rows_v)
        pltpu.sync_copy(p1_hbm.at[pl.ds(base, chunk)], i1_v)
        pltpu.sync_copy(p2_hbm.at[pl.ds(base, chunk)], i2_v)
        c1 = pltpu.async_copy(rows_v, xs_hbm.at[i1_v], sem1)
        c2 = pltpu.async_copy(rows_v, xs_hbm.at[i2_v], sem2)
        c1.wait()
        c2.wait()

    return k(x2, pos1, pos2)


def _combine_sc(ys, pos1, pos2, s):
    _, h = ys.shape
    info = plsc.get_sparse_core_info()
    nw = info.num_cores * info.num_subcores
    nl = info.num_lanes
    chunk = s // nw
    mesh = plsc.VectorSubcoreMesh(core_axis_name="c", subcore_axis_name="s")

    half = chunk // 2

    @functools.partial(
        pl.kernel, mesh=mesh,
        out_type=jax.ShapeDtypeStruct((s, h), jnp.float32),
        scratch_types=[
            pltpu.VMEM((half, h), jnp.float32),
            pltpu.VMEM((half, h), jnp.float32),
            pltpu.VMEM((half, h), jnp.float32),
            pltpu.VMEM((half, h), jnp.float32),
            pltpu.VMEM((half,), jnp.int32),
            pltpu.VMEM((half,), jnp.int32),
            pltpu.VMEM((half,), jnp.int32),
            pltpu.VMEM((half,), jnp.int32),
            pltpu.SemaphoreType.DMA,
            pltpu.SemaphoreType.DMA,
            pltpu.SemaphoreType.DMA,
        ],
    )
    def k(ys_hbm, p1_hbm, p2_hbm, out_hbm,
          r1a_v, r2a_v, r1b_v, r2b_v, i1a_v, i2a_v, i1b_v, i2b_v,
          sema, semb, semo):
        wid = lax.axis_index("s") * info.num_cores + lax.axis_index("c")
        base = wid * chunk
        pltpu.sync_copy(p1_hbm.at[pl.ds(base, half)], i1a_v)
        pltpu.sync_copy(p2_hbm.at[pl.ds(base, half)], i2a_v)
        g1a = pltpu.async_copy(ys_hbm.at[i1a_v], r1a_v, sema)
        g2a = pltpu.async_copy(ys_hbm.at[i2a_v], r2a_v, sema)
        pltpu.sync_copy(p1_hbm.at[pl.ds(base + half, half)], i1b_v)
        pltpu.sync_copy(p2_hbm.at[pl.ds(base + half, half)], i2b_v)
        g1b = pltpu.async_copy(ys_hbm.at[i1b_v], r1b_v, semb)
        g2b = pltpu.async_copy(ys_hbm.at[i2b_v], r2b_v, semb)

        def add_half(ra_v, rb_v):
            def token_body(i, _):
                for j in range(h // nl):
                    sl = pl.ds(j * nl, nl)
                    ra_v[i, sl] = ra_v[i, sl] + rb_v[i, sl]
                return 0

            lax.fori_loop(0, half, token_body, 0)

        g1a.wait()
        g2a.wait()
        add_half(r1a_v, r2a_v)
        sta = pltpu.async_copy(r1a_v, out_hbm.at[pl.ds(base, half)], semo)
        g1b.wait()
        g2b.wait()
        add_half(r1b_v, r2b_v)
        stb = pltpu.async_copy(r1b_v, out_hbm.at[pl.ds(base + half, half)],
                               semo)
        sta.wait()
        stb.wait()

    return k(ys, pos1, pos2)


def kernel(x, gate_W, gate_b, W1, b1, W2, b2):
    b, s, h = x.shape
    e = W1.shape[0]
    x2 = x.reshape(b * s, h)
    st = b * s

    e1, e2, w1, w2, r1, r2, counts = _gating(x2, gate_W, gate_b)

    nt_e = (counts + TOKEN_TILE - 1) // TOKEN_TILE
    cum = jnp.cumsum(nt_e)
    ntiles = cum[-1]
    tile_base = (cum - nt_e) * TOKEN_TILE
    max_tiles = (2 * st) // TOKEN_TILE + e - 1
    tq = jnp.minimum(jnp.arange(max_tiles, dtype=jnp.int32), ntiles - 1)
    tile_expert = jnp.searchsorted(cum, tq, side="right").astype(jnp.int32)
    pos1 = tile_base[e1] + r1
    pos2 = tile_base[e2] + r2
    num_rows = max_tiles * TOKEN_TILE
    gates = (jnp.zeros((num_rows,), jnp.float32)
             .at[pos1].set(w1).at[pos2].set(w2))

    xs = _dispatch_sc(x2, pos1, pos2, num_rows)
    ys = _grouped_experts(xs, W1, b1, W2, b2, gates, tile_expert,
                          ntiles.reshape(1).astype(jnp.int32))
    out = _combine_sc(ys, pos1, pos2, st)
    return out.reshape(b, s, h)
